# baseline (speedup 1.0000x reference)
import jax
import jax.numpy as jnp
from jax.experimental import pallas as pl
from jax.experimental.pallas import tpu as pltpu

T = 10000
L = 128
S = 5
OUT = 8
TILE = 10000
NT = T // TILE
HALF = TILE // 2

_BF = jnp.bfloat16
_F32 = jnp.float32


def _dot(x, w):
    return jax.lax.dot_general(
        x.astype(_BF), w.astype(_BF),
        (((x.ndim - 1,), (0,)), ((), ())),
        preferred_element_type=_F32)


def _bmul(a, b):
    return a.astype(_BF).astype(_F32) * b.astype(_BF).astype(_F32)


def _norm(x):
    return jnp.sqrt(jnp.sum(x * x, axis=-1, keepdims=True))


def _relu_bf(x):
    return jnp.maximum(x.astype(_BF), jnp.zeros((), _BF))


def _body(sav_r, feat_r,
          esW0, esb0, esW1, esb1, esW2, esb2,
          etW0, etb0, etW1, etb1, etW2, etb2,
          feW0, fnrow, feb0, feW1, feb1, feW2, feb2,
          fW0, fhsW0, fb0, fW1, fb1, fW2, fb2,
          psW0, psb0, psW1, psb1, psW2, psb2,
          ptW0, ptb0, ptW1, ptb1, ptW2, ptb2,
          dW0, db0, dW1, db1, dW2, db2,
          out_r,
          hest_s, hets_s, htgt_s, hself_s, acc_s):
    relu = jax.nn.relu
    p = pl.program_id(0)
    t = pl.program_id(1)
    rows = pl.ds(t * TILE, TILE)

    @pl.when(p == 0)
    def _encode():
        feat = feat_r[...]
        trv = feat[:, 0:3]
        tav = feat[:, 3:6]
        trd = feat[:, 6:9]
        nr = jnp.sqrt(jnp.concatenate(
            [jnp.sum(trv * trv, axis=-1, keepdims=True),
             jnp.sum(tav * tav, axis=-1, keepdims=True),
             jnp.sum(trd * trd, axis=-1, keepdims=True)], axis=1))
        nrm_trv = nr[:, 0:1]
        nrm_tav = nr[:, 1:2]
        nrm_trd = nr[:, 2:3]

        w = etW0[...]
        x = _relu_bf(_dot(trv, w[0:3]) + _bmul(nrm_trv, w[3:4])
                     + _dot(tav, w[4:7]) + _bmul(nrm_tav, w[7:8]) + etb0[...])
        x = _relu_bf(_dot(x, etW1[...]) + etb1[...])
        htgt_s[rows, :] = _dot(x, etW2[...]) + etb2[...]

        x = _relu_bf(_dot(trd, feW0[...]) + _bmul(nrm_trd, fnrow[...])
                     + feb0[...])
        x = _relu_bf(_dot(x, feW1[...]) + feb1[...])
        y = _dot(x, feW2[...]) + feb2[...]
        hest_s[rows, :] = y[:, 0:L]
        hets_s[rows, :] = y[:, L:2 * L]

        @pl.when(t == 0)
        def _enc_self():
            sav = sav_r[...]
            w = esW0[...]
            x = _relu_bf(_dot(sav, w[0:3]) + _bmul(_norm(sav), w[3:4]) + esb0[...])
            x = _relu_bf(_dot(x, esW1[...]) + esb1[...])
            hself_s[...] = _dot(x, esW2[...]) + esb2[...]

    @pl.when(p > 0)
    def _process():
        s = p - 1
        hs = hself_s[...]
        bias = _dot(hs, fhsW0[s]) + fb0[s]

        def half(r):
            he = hest_s[r, :]
            hx = hets_s[r, :]
            ht = htgt_s[r, :]
            heb = he.astype(_BF)
            hxb = hx.astype(_BF)
            htb = ht.astype(_BF)

            u = jnp.concatenate([heb, hxb, htb], axis=1)
            x = _relu_bf(_dot(u, fW0[s]) + bias)
            x = _relu_bf(_dot(x, fW1[s]) + fb1[s])
            d = _dot(x, fW2[s]) + fb2[s]
            new_est = he + d[:, 0:L]
            new_ets = hx + d[:, L:2 * L]

            v = jnp.concatenate([htb, new_est.astype(_BF)], axis=1)
            x = _relu_bf(_dot(v, ptW0[s]) + ptb0[s])
            x = _relu_bf(_dot(x, ptW1[s]) + ptb1[s])
            htgt_s[r, :] = ht + _dot(x, ptW2[s]) + ptb2[s]

            hest_s[r, :] = new_est
            hets_s[r, :] = new_ets
            return new_ets

        ets0 = half(pl.ds(t * TILE, HALF))
        ets1 = half(pl.ds(t * TILE + HALF, HALF))
        part = jnp.sum(jnp.concatenate([ets0, ets1], axis=0),
                       axis=0, keepdims=True)

        @pl.when(t == 0)
        def _():
            acc_s[...] = part

        @pl.when(t > 0)
        def _():
            acc_s[...] += part

        @pl.when(t == NT - 1)
        def _self_update():
            agg = acc_s[...]
            w0 = psW0[s]
            x = _relu_bf(_dot(hs, w0[0:L]) + _dot(agg, w0[L:2 * L]) + psb0[s])
            x = _relu_bf(_dot(x, psW1[s]) + psb1[s])
            hs_new = hs + _dot(x, psW2[s]) + psb2[s]
            hself_s[...] = hs_new

            @pl.when(p == S)
            def _decode():
                y = _relu_bf(_dot(hs_new, dW0[...]) + db0[...])
                y = _relu_bf(_dot(y, dW1[...]) + db1[...])
                out_r[...] = _dot(y, dW2[...]) + db2[...]


def _full(shape):
    nd = len(shape)
    return pl.BlockSpec(shape, lambda p, t, _n=nd: (0,) * _n)


def kernel(self_angular_velocity, target_relative_velocity, target_angular_velocity, target_relative_displacement, enc_self_W0, enc_self_b0, enc_self_W1, enc_self_b1, enc_self_W2, enc_self_b2, enc_tgt_W0, enc_tgt_b0, enc_tgt_W1, enc_tgt_b1, enc_tgt_W2, enc_tgt_b2, enc_est_W0, enc_est_b0, enc_est_W1, enc_est_b1, enc_est_W2, enc_est_b2, enc_ets_W0, enc_ets_b0, enc_ets_W1, enc_ets_b1, enc_ets_W2, enc_ets_b2, proc_est_W0, proc_est_b0, proc_est_W1, proc_est_b1, proc_est_W2, proc_est_b2, proc_ets_W0, proc_ets_b0, proc_ets_W1, proc_ets_b1, proc_ets_W2, proc_ets_b2, proc_self_W0, proc_self_b0, proc_self_W1, proc_self_b1, proc_self_W2, proc_self_b2, proc_tgt_W0, proc_tgt_b0, proc_tgt_W1, proc_tgt_b1, proc_tgt_W2, proc_tgt_b2, dec_self_W0, dec_self_b0, dec_self_W1, dec_self_b1, dec_self_W2, dec_self_b2):
    bf = lambda a: a.astype(_BF)
    r2 = lambda b: b.reshape(1, -1)
    r3 = lambda b: b.reshape(S, 1, -1)
    packed = jnp.concatenate([target_relative_velocity,
                              target_angular_velocity,
                              target_relative_displacement], axis=1)

    z1 = jnp.zeros((L, L), _F32)
    feW0 = jnp.concatenate([enc_est_W0[0:3], -enc_ets_W0[0:3]], axis=1)
    fnrow = jnp.concatenate([enc_est_W0[3:4], enc_ets_W0[3:4]], axis=1)
    feb0 = jnp.concatenate([enc_est_b0, enc_ets_b0]).reshape(1, -1)
    feW1 = jnp.concatenate([
        jnp.concatenate([enc_est_W1, z1], axis=1),
        jnp.concatenate([z1, enc_ets_W1], axis=1)], axis=0)
    feb1 = jnp.concatenate([enc_est_b1, enc_ets_b1]).reshape(1, -1)
    feW2 = jnp.concatenate([
        jnp.concatenate([enc_est_W2, z1], axis=1),
        jnp.concatenate([z1, enc_ets_W2], axis=1)], axis=0)
    feb2 = jnp.concatenate([enc_est_b2, enc_ets_b2]).reshape(1, -1)

    We0, Wx0 = proc_est_W0, proc_ets_W0
    z = jnp.zeros((S, L, L), _F32)
    fW0 = jnp.concatenate([
        jnp.concatenate([We0[:, 0:L], z], axis=2),
        jnp.concatenate([z, Wx0[:, 0:L]], axis=2),
        jnp.concatenate([We0[:, 2 * L:3 * L], Wx0[:, L:2 * L]], axis=2),
    ], axis=1)
    fhsW0 = jnp.concatenate([We0[:, L:2 * L], Wx0[:, 2 * L:3 * L]], axis=2)
    fb0 = jnp.concatenate([proc_est_b0, proc_ets_b0], axis=1)
    fW1 = jnp.concatenate([
        jnp.concatenate([proc_est_W1, z], axis=2),
        jnp.concatenate([z, proc_ets_W1], axis=2),
    ], axis=1)
    fb1 = jnp.concatenate([proc_est_b1, proc_ets_b1], axis=1)
    fW2 = jnp.concatenate([
        jnp.concatenate([proc_est_W2, z], axis=2),
        jnp.concatenate([z, proc_ets_W2], axis=2),
    ], axis=1)
    fb2 = jnp.concatenate([proc_est_b2, proc_ets_b2], axis=1)

    args = (
        self_angular_velocity, packed,
        bf(enc_self_W0), r2(enc_self_b0), bf(enc_self_W1), r2(enc_self_b1), bf(enc_self_W2), r2(enc_self_b2),
        bf(enc_tgt_W0), r2(enc_tgt_b0), bf(enc_tgt_W1), r2(enc_tgt_b1), bf(enc_tgt_W2), r2(enc_tgt_b2),
        bf(feW0), bf(fnrow), feb0, bf(feW1), feb1, bf(feW2), feb2,
        bf(fW0), bf(fhsW0), r3(fb0), bf(fW1), r3(fb1), bf(fW2), r3(fb2),
        bf(proc_self_W0), r3(proc_self_b0), bf(proc_self_W1), r3(proc_self_b1), bf(proc_self_W2), r3(proc_self_b2),
        bf(proc_tgt_W0), r3(proc_tgt_b0), bf(proc_tgt_W1), r3(proc_tgt_b1), bf(proc_tgt_W2), r3(proc_tgt_b2),
        bf(dec_self_W0), r2(dec_self_b0), bf(dec_self_W1), r2(dec_self_b1), bf(dec_self_W2), r2(dec_self_b2),
    )
    in_specs = [
        _full((1, 3)),
        pl.BlockSpec((TILE, 9), lambda p, t: (t, 0)),
    ] + [_full(a.shape) for a in args[2:]]
    return pl.pallas_call(
        _body,
        grid=(S + 1, NT),
        in_specs=in_specs,
        out_specs=pl.BlockSpec((1, OUT), lambda p, t: (0, 0)),
        out_shape=jax.ShapeDtypeStruct((1, OUT), _F32),
        scratch_shapes=[
            pltpu.VMEM((T, L), _F32),
            pltpu.VMEM((T, L), _F32),
            pltpu.VMEM((T, L), _F32),
            pltpu.VMEM((1, L), _F32),
            pltpu.VMEM((1, L), _F32),
        ],
        compiler_params=pltpu.CompilerParams(
            dimension_semantics=("arbitrary", "arbitrary"),
            vmem_limit_bytes=60000 * 1024,
        ),
    )(*args)


# speedup vs baseline: 1.0014x; 1.0014x over previous
import jax
import jax.numpy as jnp
from jax.experimental import pallas as pl
from jax.experimental.pallas import tpu as pltpu

T = 10000
L = 128
S = 5
OUT = 8
TILE = 10000
NT = T // TILE
HALF = TILE // 2

_BF = jnp.bfloat16
_F32 = jnp.float32


def _dot(x, w):
    return jax.lax.dot_general(
        x.astype(_BF), w.astype(_BF),
        (((x.ndim - 1,), (0,)), ((), ())),
        preferred_element_type=_F32)


def _bmul(a, b):
    return a.astype(_BF).astype(_F32) * b.astype(_BF).astype(_F32)


def _norm(x):
    return jnp.sqrt(jnp.sum(x * x, axis=-1, keepdims=True))


def _relu_bf(x):
    return jnp.maximum(x.astype(_BF), jnp.zeros((), _BF))


def _body(sav_r, feat_r,
          esW0, esb0, esW1, esb1, esW2, esb2,
          etW0, etb0, etW1, etb1, etW2, etb2,
          feW0, fnrow, feb0, feW1, feb1, feW2, feb2,
          fW0, fhsW0, fb0, fW1, fb1, fW2, fb2,
          psW0, psb0, psW1, psb1, psW2, psb2,
          ptW0, ptb0, ptW1, ptb1, ptW2, ptb2,
          dW0, db0, dW1, db1, dW2, db2,
          out_r,
          hest_s, hets_s, htgt_s, hself_s, acc_s, hallb_s):
    relu = jax.nn.relu
    p = pl.program_id(0)
    t = pl.program_id(1)
    rows = pl.ds(t * TILE, TILE)

    @pl.when(p == 0)
    def _encode():
        feat = feat_r[...]
        trv = feat[:, 0:3]
        tav = feat[:, 3:6]
        trd = feat[:, 6:9]
        nr = jnp.sqrt(jnp.concatenate(
            [jnp.sum(trv * trv, axis=-1, keepdims=True),
             jnp.sum(tav * tav, axis=-1, keepdims=True),
             jnp.sum(trd * trd, axis=-1, keepdims=True)], axis=1))
        nrm_trv = nr[:, 0:1]
        nrm_tav = nr[:, 1:2]
        nrm_trd = nr[:, 2:3]

        w = etW0[...]
        x = _relu_bf(_dot(trv, w[0:3]) + _bmul(nrm_trv, w[3:4])
                     + _dot(tav, w[4:7]) + _bmul(nrm_tav, w[7:8]) + etb0[...])
        x = _relu_bf(_dot(x, etW1[...]) + etb1[...])
        ht0 = _dot(x, etW2[...]) + etb2[...]
        htgt_s[rows, :] = ht0

        x = _relu_bf(_dot(trd, feW0[...]) + _bmul(nrm_trd, fnrow[...])
                     + feb0[...])
        x = _relu_bf(_dot(x, feW1[...]) + feb1[...])
        y = _dot(x, feW2[...]) + feb2[...]
        hest_s[rows, :] = y[:, 0:L]
        hets_s[rows, :] = y[:, L:2 * L]
        hallb_s[rows, 0:2 * L] = y.astype(_BF)
        hallb_s[rows, 2 * L:3 * L] = ht0.astype(_BF)

        @pl.when(t == 0)
        def _enc_self():
            sav = sav_r[...]
            w = esW0[...]
            x = _relu_bf(_dot(sav, w[0:3]) + _bmul(_norm(sav), w[3:4]) + esb0[...])
            x = _relu_bf(_dot(x, esW1[...]) + esb1[...])
            hself_s[...] = _dot(x, esW2[...]) + esb2[...]

    @pl.when(p > 0)
    def _process():
        s = p - 1
        hs = hself_s[...]
        bias = _dot(hs, fhsW0[s]) + fb0[s]

        def half(r):
            he = hest_s[r, :]
            hx = hets_s[r, :]
            ht = htgt_s[r, :]

            u = hallb_s[r, :]
            x = _relu_bf(_dot(u, fW0[s]) + bias)
            x = _relu_bf(_dot(x, fW1[s]) + fb1[s])
            d = _dot(x, fW2[s]) + fb2[s]
            new_est = he + d[:, 0:L]
            new_ets = hx + d[:, L:2 * L]
            neb = new_est.astype(_BF)

            v = jnp.concatenate([u[:, 2 * L:3 * L], neb], axis=1)
            x = _relu_bf(_dot(v, ptW0[s]) + ptb0[s])
            x = _relu_bf(_dot(x, ptW1[s]) + ptb1[s])
            tgt_new = ht + _dot(x, ptW2[s]) + ptb2[s]
            htgt_s[r, :] = tgt_new

            hest_s[r, :] = new_est
            hets_s[r, :] = new_ets
            hallb_s[r, 0:L] = neb
            hallb_s[r, L:2 * L] = new_ets.astype(_BF)
            hallb_s[r, 2 * L:3 * L] = tgt_new.astype(_BF)
            return new_ets

        ets0 = half(pl.ds(t * TILE, HALF))
        ets1 = half(pl.ds(t * TILE + HALF, HALF))
        part = jnp.sum(jnp.concatenate([ets0, ets1], axis=0),
                       axis=0, keepdims=True)

        @pl.when(t == 0)
        def _():
            acc_s[...] = part

        @pl.when(t > 0)
        def _():
            acc_s[...] += part

        @pl.when(t == NT - 1)
        def _self_update():
            agg = acc_s[...]
            w0 = psW0[s]
            x = _relu_bf(_dot(hs, w0[0:L]) + _dot(agg, w0[L:2 * L]) + psb0[s])
            x = _relu_bf(_dot(x, psW1[s]) + psb1[s])
            hs_new = hs + _dot(x, psW2[s]) + psb2[s]
            hself_s[...] = hs_new

            @pl.when(p == S)
            def _decode():
                y = _relu_bf(_dot(hs_new, dW0[...]) + db0[...])
                y = _relu_bf(_dot(y, dW1[...]) + db1[...])
                out_r[...] = _dot(y, dW2[...]) + db2[...]


def _full(shape):
    nd = len(shape)
    return pl.BlockSpec(shape, lambda p, t, _n=nd: (0,) * _n)


def kernel(self_angular_velocity, target_relative_velocity, target_angular_velocity, target_relative_displacement, enc_self_W0, enc_self_b0, enc_self_W1, enc_self_b1, enc_self_W2, enc_self_b2, enc_tgt_W0, enc_tgt_b0, enc_tgt_W1, enc_tgt_b1, enc_tgt_W2, enc_tgt_b2, enc_est_W0, enc_est_b0, enc_est_W1, enc_est_b1, enc_est_W2, enc_est_b2, enc_ets_W0, enc_ets_b0, enc_ets_W1, enc_ets_b1, enc_ets_W2, enc_ets_b2, proc_est_W0, proc_est_b0, proc_est_W1, proc_est_b1, proc_est_W2, proc_est_b2, proc_ets_W0, proc_ets_b0, proc_ets_W1, proc_ets_b1, proc_ets_W2, proc_ets_b2, proc_self_W0, proc_self_b0, proc_self_W1, proc_self_b1, proc_self_W2, proc_self_b2, proc_tgt_W0, proc_tgt_b0, proc_tgt_W1, proc_tgt_b1, proc_tgt_W2, proc_tgt_b2, dec_self_W0, dec_self_b0, dec_self_W1, dec_self_b1, dec_self_W2, dec_self_b2):
    bf = lambda a: a.astype(_BF)
    r2 = lambda b: b.reshape(1, -1)
    r3 = lambda b: b.reshape(S, 1, -1)
    packed = jnp.concatenate([target_relative_velocity,
                              target_angular_velocity,
                              target_relative_displacement], axis=1)

    z1 = jnp.zeros((L, L), _F32)
    feW0 = jnp.concatenate([enc_est_W0[0:3], -enc_ets_W0[0:3]], axis=1)
    fnrow = jnp.concatenate([enc_est_W0[3:4], enc_ets_W0[3:4]], axis=1)
    feb0 = jnp.concatenate([enc_est_b0, enc_ets_b0]).reshape(1, -1)
    feW1 = jnp.concatenate([
        jnp.concatenate([enc_est_W1, z1], axis=1),
        jnp.concatenate([z1, enc_ets_W1], axis=1)], axis=0)
    feb1 = jnp.concatenate([enc_est_b1, enc_ets_b1]).reshape(1, -1)
    feW2 = jnp.concatenate([
        jnp.concatenate([enc_est_W2, z1], axis=1),
        jnp.concatenate([z1, enc_ets_W2], axis=1)], axis=0)
    feb2 = jnp.concatenate([enc_est_b2, enc_ets_b2]).reshape(1, -1)

    We0, Wx0 = proc_est_W0, proc_ets_W0
    z = jnp.zeros((S, L, L), _F32)
    fW0 = jnp.concatenate([
        jnp.concatenate([We0[:, 0:L], z], axis=2),
        jnp.concatenate([z, Wx0[:, 0:L]], axis=2),
        jnp.concatenate([We0[:, 2 * L:3 * L], Wx0[:, L:2 * L]], axis=2),
    ], axis=1)
    fhsW0 = jnp.concatenate([We0[:, L:2 * L], Wx0[:, 2 * L:3 * L]], axis=2)
    fb0 = jnp.concatenate([proc_est_b0, proc_ets_b0], axis=1)
    fW1 = jnp.concatenate([
        jnp.concatenate([proc_est_W1, z], axis=2),
        jnp.concatenate([z, proc_ets_W1], axis=2),
    ], axis=1)
    fb1 = jnp.concatenate([proc_est_b1, proc_ets_b1], axis=1)
    fW2 = jnp.concatenate([
        jnp.concatenate([proc_est_W2, z], axis=2),
        jnp.concatenate([z, proc_ets_W2], axis=2),
    ], axis=1)
    fb2 = jnp.concatenate([proc_est_b2, proc_ets_b2], axis=1)

    args = (
        self_angular_velocity, packed,
        bf(enc_self_W0), r2(enc_self_b0), bf(enc_self_W1), r2(enc_self_b1), bf(enc_self_W2), r2(enc_self_b2),
        bf(enc_tgt_W0), r2(enc_tgt_b0), bf(enc_tgt_W1), r2(enc_tgt_b1), bf(enc_tgt_W2), r2(enc_tgt_b2),
        bf(feW0), bf(fnrow), feb0, bf(feW1), feb1, bf(feW2), feb2,
        bf(fW0), bf(fhsW0), r3(fb0), bf(fW1), r3(fb1), bf(fW2), r3(fb2),
        bf(proc_self_W0), r3(proc_self_b0), bf(proc_self_W1), r3(proc_self_b1), bf(proc_self_W2), r3(proc_self_b2),
        bf(proc_tgt_W0), r3(proc_tgt_b0), bf(proc_tgt_W1), r3(proc_tgt_b1), bf(proc_tgt_W2), r3(proc_tgt_b2),
        bf(dec_self_W0), r2(dec_self_b0), bf(dec_self_W1), r2(dec_self_b1), bf(dec_self_W2), r2(dec_self_b2),
    )
    in_specs = [
        _full((1, 3)),
        pl.BlockSpec((TILE, 9), lambda p, t: (t, 0)),
    ] + [_full(a.shape) for a in args[2:]]
    return pl.pallas_call(
        _body,
        grid=(S + 1, NT),
        in_specs=in_specs,
        out_specs=pl.BlockSpec((1, OUT), lambda p, t: (0, 0)),
        out_shape=jax.ShapeDtypeStruct((1, OUT), _F32),
        scratch_shapes=[
            pltpu.VMEM((T, L), _F32),
            pltpu.VMEM((T, L), _F32),
            pltpu.VMEM((T, L), _F32),
            pltpu.VMEM((1, L), _F32),
            pltpu.VMEM((1, L), _F32),
            pltpu.VMEM((T, 3 * L), _BF),
        ],
        compiler_params=pltpu.CompilerParams(
            dimension_semantics=("arbitrary", "arbitrary"),
            vmem_limit_bytes=60000 * 1024,
        ),
    )(*args)
